# baseline (speedup 1.0000x reference)
import functools

import jax
import jax.numpy as jnp
from jax import lax
from jax.experimental import pallas as pl
from jax.experimental.pallas import tpu as pltpu
from jax.experimental.pallas import tpu_sc as plsc

N = 10000
E = 320000
D = 128

NC = 2
NS = 16

ZROWS = 80
NBLOCKS = N // ZROWS

CH2 = 125
ROWS2 = E // CH2
TCH2 = ROWS2 // NS
GCH2 = 16
NG2 = TCH2 // GCH2

CH4 = 80
EPT4 = E // (NC * NS)
TCH4 = EPT4 // CH4


def _bn(x, g, b, eps=1e-5):
    mu = jnp.mean(x, axis=0, keepdims=True)
    var = jnp.mean((x - mu) ** 2, axis=0, keepdims=True)
    return (x - mu) / jnp.sqrt(var + eps) * g + b


def _mlp_bn_relu(x, W1, b1, g1, be1, W2, b2, go, bo):
    h = lax.dot_general(x, W1, (((1,), (1,)), ((), ())),
                        preferred_element_type=jnp.float32) + b1
    h = jax.nn.relu(_bn(h, g1, be1))
    y = lax.dot_general(h, W2, (((1,), (1,)), ((), ())),
                        preferred_element_type=jnp.float32) + b2
    return jax.nn.relu(_bn(y, go, bo))


def _stage1_body(feat, *refs):
    (w1s, b1s, g1s, be1s, w2s, b2s, gs, bs,
     w1d, b1d, g1d, be1d, w2d, b2d, gd, bd, srcf, dstf, zref) = refs
    x = feat[...]
    srcf[...] = _mlp_bn_relu(x, w1s[...], b1s[...], g1s[...], be1s[...],
                             w2s[...], b2s[...], gs[...], bs[...])
    dstf[...] = _mlp_bn_relu(x, w1d[...], b1d[...], g1d[...], be1d[...],
                             w2d[...], b2d[...], gd[...], bd[...])
    zref[...] = jnp.zeros_like(x)


def _stage3_body(aggA, aggB, *refs):
    (w1r, b1r, g1r, be1r, w2r, b2r, g3, b3,
     w1s, b1s, g1s, be1s, w2s, b2s, g4, b4,
     w1d, b1d, g1d, be1d, w2d, b2d, g5, b5, cat2) = refs
    h = (lax.dot_general(aggA[...], w1r[:, :D], (((1,), (1,)), ((), ())),
                         preferred_element_type=jnp.float32)
         + lax.dot_general(aggB[...], w1r[:, D:], (((1,), (1,)), ((), ())),
                           preferred_element_type=jnp.float32) + b1r[...])
    h = jax.nn.relu(_bn(h, g1r[...], be1r[...]))
    y = lax.dot_general(h, w2r[...], (((1,), (1,)), ((), ())),
                        preferred_element_type=jnp.float32) + b2r[...]
    node = jax.nn.relu(_bn(y, g3[...], b3[...]))
    cat2[:, :D] = _mlp_bn_relu(node, w1s[...], b1s[...], g1s[...], be1s[...],
                               w2s[...], b2s[...], g4[...], b4[...])
    cat2[:, D:] = _mlp_bn_relu(node, w1d[...], b1d[...], g1d[...], be1d[...],
                               w2d[...], b2d[...], g5[...], b5[...])


def _pipelined(nchunks, start, drain, bufs):
    nbuf = len(bufs)
    assert nchunks % nbuf == 0
    for b in range(nbuf - 1):
        start(b, bufs[b])

    def body(g, _):
        j0 = g * nbuf
        for b in range(nbuf):
            i = j0 + b

            @pl.when(i + nbuf - 1 < nchunks)
            def _():
                start(i + nbuf - 1, bufs[(b + nbuf - 1) % nbuf])

            drain(i, bufs[b])
        return 0

    lax.fori_loop(0, nchunks // nbuf, body, 0)


_SC_MESH = plsc.VectorSubcoreMesh(core_axis_name="c", subcore_axis_name="s")


@functools.partial(
    pl.kernel,
    out_type=(jax.ShapeDtypeStruct((N, D), jnp.float32),
              jax.ShapeDtypeStruct((N, D), jnp.float32)),
    mesh=_SC_MESH,
    scratch_types=[
        pltpu.VMEM((GCH2, CH2), jnp.int32),
        pltpu.VMEM((GCH2, CH2), jnp.int32),
        pltpu.VMEM((GCH2, CH2), jnp.int32),
        pltpu.VMEM((GCH2, CH2), jnp.int32),
        pltpu.VMEM((CH2, D), jnp.float32),
        pltpu.VMEM((CH2, D), jnp.float32),
        pltpu.VMEM_SHARED((N, D), jnp.float32),
        pltpu.SemaphoreType.DMA,
        pltpu.SemaphoreType.DMA,
        pltpu.SemaphoreType.DMA,
        pltpu.SemaphoreType.DMA,
    ],
)
def _segment_sum_sc(srcf_hbm, dstf_hbm, zeros_hbm, src2d_hbm, dst2d_hbm,
                    aggA_hbm, aggB_hbm,
                    idxs0, idxd0, idxs1, idxd1, rows0, rows1, acc_sh,
                    sem0, sem1, isem0, isem1):
    c = lax.axis_index("c")
    s = lax.axis_index("s")
    zbuf_v = rows1.at[pl.ds(0, ZROWS)]

    def over_blocks(fn):
        def body(j, _):
            blk = s + j * NS

            @pl.when(blk < NBLOCKS)
            def _():
                fn(blk * ZROWS)

            return 0

        lax.fori_loop(0, (NBLOCKS + NS - 1) // NS, body, 0)

    idx_sets = ((idxs0, idxd0, isem0), (idxs1, idxd1, isem1))

    def idx_copies(g, iset):
        idxs, idxd, isem = iset
        base = s * TCH2 + g * GCH2
        return (pltpu.make_async_copy(
                    src2d_hbm.at[pl.ds(base, GCH2)], idxs, isem),
                pltpu.make_async_copy(
                    dst2d_hbm.at[pl.ds(base, GCH2)], idxd, isem))

    def load_idx(g, iset):
        for cp in idx_copies(g, iset):
            cp.start()

    def wait_idx(g, iset):
        for cp in idx_copies(g, iset):
            cp.wait()

    def run(table_hbm):
        def run_group(iset):
            idxs, idxd, _ = iset

            def start(i, bufsem):
                buf, sem = bufsem
                pltpu.async_copy(table_hbm.at[idxs.at[i]], buf, sem)

            def drain(i, bufsem):
                buf, sem = bufsem
                pltpu.make_async_copy(
                    table_hbm.at[idxs.at[i]], buf, sem).wait()
                pltpu.sync_copy(buf, acc_sh.at[idxd.at[i]], add=True)

            _pipelined(GCH2, start, drain, ((rows0, sem0), (rows1, sem1)))

        load_idx(0, idx_sets[0])

        def zinit(base):
            pltpu.sync_copy(zeros_hbm.at[pl.ds(base, ZROWS)], zbuf_v)
            pltpu.sync_copy(zbuf_v, acc_sh.at[pl.ds(base, ZROWS)])

        over_blocks(zinit)
        plsc.subcore_barrier()

        def group(g, _):
            for par in (0, 1):
                @pl.when(lax.rem(g, 2) == par)
                def _():
                    wait_idx(g, idx_sets[par])

                    @pl.when(g + 1 < NG2)
                    def _():
                        load_idx(g + 1, idx_sets[1 - par])

                    run_group(idx_sets[par])

            return 0

        lax.fori_loop(0, NG2, group, 0)

    @pl.when(c == 0)
    def _():
        run(srcf_hbm)

    @pl.when(c == 1)
    def _():
        run(dstf_hbm)

    plsc.subcore_barrier()

    def wb(out_hbm):
        def step(base):
            pltpu.sync_copy(acc_sh.at[pl.ds(base, ZROWS)], zbuf_v)
            pltpu.sync_copy(zbuf_v, out_hbm.at[pl.ds(base, ZROWS)])

        over_blocks(step)

    @pl.when(c == 0)
    def _():
        wb(aggA_hbm)

    @pl.when(c == 1)
    def _():
        wb(aggB_hbm)


@functools.partial(
    pl.kernel,
    out_type=jax.ShapeDtypeStruct((E, 2 * D), jnp.float32),
    mesh=_SC_MESH,
    scratch_types=[
        pltpu.VMEM((EPT4,), jnp.int32),
    ] + [pltpu.VMEM((CH4, 2 * D), jnp.float32)] * 5
      + [pltpu.SemaphoreType.DMA] * 5,
)
def _edge_gather_sc(cat2_hbm, src_hbm, out_hbm, idx_v, *bufsems):
    c = lax.axis_index("c")
    s = lax.axis_index("s")
    wid = s * NC + c
    e0 = wid * EPT4

    pltpu.sync_copy(src_hbm.at[pl.ds(e0, EPT4)], idx_v)

    def start(i, bufsem):
        buf, sem = bufsem
        pltpu.async_copy(cat2_hbm.at[idx_v.at[pl.ds(i * CH4, CH4)]], buf, sem)

    def drain(i, bufsem):
        buf, sem = bufsem
        pltpu.make_async_copy(
            cat2_hbm.at[idx_v.at[pl.ds(i * CH4, CH4)]], buf, sem).wait()
        pltpu.sync_copy(buf, out_hbm.at[pl.ds(e0 + i * CH4, CH4)])

    rows = bufsems[:5]
    sems = bufsems[5:]
    _pipelined(TCH4, start, drain, tuple(zip(rows, sems)))


def _mlp_args(p):
    return (p["W1"], p["b1"].reshape(1, D), p["g1"].reshape(1, D),
            p["be1"].reshape(1, D), p["W2"], p["b2"].reshape(1, D))


def kernel(feat, edge_index, params):
    src = edge_index[0]
    dst = edge_index[1]
    bn = params["bn"]

    def bnv(i):
        return (bn["g%d" % i].reshape(1, D), bn["b%d" % i].reshape(1, D))

    stage1_in = (feat, *_mlp_args(params["src1"]), *bnv(1),
                 *_mlp_args(params["dst1"]), *bnv(2))
    srcf, dstf, zeros = pl.pallas_call(
        _stage1_body,
        out_shape=(jax.ShapeDtypeStruct((N, D), jnp.float32),
                   jax.ShapeDtypeStruct((N, D), jnp.float32),
                   jax.ShapeDtypeStruct((N, D), jnp.float32)),
    )(*stage1_in)

    src2d = src.reshape(ROWS2, CH2)
    dst2d = dst.reshape(ROWS2, CH2)
    aggA, aggB = _segment_sum_sc(srcf, dstf, zeros, src2d, dst2d)

    pr = params["res"]
    stage3_in = (aggA, aggB, pr["W1"], pr["b1"].reshape(1, D),
                 pr["g1"].reshape(1, D), pr["be1"].reshape(1, D),
                 pr["W2"], pr["b2"].reshape(1, D), *bnv(3),
                 *_mlp_args(params["src2"]), *bnv(4),
                 *_mlp_args(params["dst2"]), *bnv(5))
    cat2 = pl.pallas_call(
        _stage3_body,
        out_shape=jax.ShapeDtypeStruct((N, 2 * D), jnp.float32),
    )(*stage3_in)

    return _edge_gather_sc(cat2, src)


# speedup vs baseline: 1.0124x; 1.0124x over previous
import functools

import jax
import jax.numpy as jnp
from jax import lax
from jax.experimental import pallas as pl
from jax.experimental.pallas import tpu as pltpu
from jax.experimental.pallas import tpu_sc as plsc

N = 10000
E = 320000
D = 128

NC = 2
NS = 16

ZROWS = 80
NBLOCKS = N // ZROWS

CH2 = 125
ROWS2 = E // CH2
TCH2 = ROWS2 // NS
GCH2 = 32
NG2 = TCH2 // GCH2

CH4 = 80
EPT4 = E // (NC * NS)
TCH4 = EPT4 // CH4


def _bn(x, g, b, eps=1e-5):
    mu = jnp.mean(x, axis=0, keepdims=True)
    var = jnp.mean((x - mu) ** 2, axis=0, keepdims=True)
    return (x - mu) / jnp.sqrt(var + eps) * g + b


def _mlp_bn_relu(x, W1, b1, g1, be1, W2, b2, go, bo):
    h = lax.dot_general(x, W1, (((1,), (1,)), ((), ())),
                        preferred_element_type=jnp.float32) + b1
    h = jax.nn.relu(_bn(h, g1, be1))
    y = lax.dot_general(h, W2, (((1,), (1,)), ((), ())),
                        preferred_element_type=jnp.float32) + b2
    return jax.nn.relu(_bn(y, go, bo))


def _stage1_body(feat, *refs):
    (w1s, b1s, g1s, be1s, w2s, b2s, gs, bs,
     w1d, b1d, g1d, be1d, w2d, b2d, gd, bd, srcf, dstf, zref) = refs
    x = feat[...]
    srcf[...] = _mlp_bn_relu(x, w1s[...], b1s[...], g1s[...], be1s[...],
                             w2s[...], b2s[...], gs[...], bs[...])
    dstf[...] = _mlp_bn_relu(x, w1d[...], b1d[...], g1d[...], be1d[...],
                             w2d[...], b2d[...], gd[...], bd[...])
    zref[...] = jnp.zeros_like(x)


def _stage3_body(aggA, aggB, *refs):
    (w1r, b1r, g1r, be1r, w2r, b2r, g3, b3,
     w1s, b1s, g1s, be1s, w2s, b2s, g4, b4,
     w1d, b1d, g1d, be1d, w2d, b2d, g5, b5, cat2) = refs
    h = (lax.dot_general(aggA[...], w1r[:, :D], (((1,), (1,)), ((), ())),
                         preferred_element_type=jnp.float32)
         + lax.dot_general(aggB[...], w1r[:, D:], (((1,), (1,)), ((), ())),
                           preferred_element_type=jnp.float32) + b1r[...])
    h = jax.nn.relu(_bn(h, g1r[...], be1r[...]))
    y = lax.dot_general(h, w2r[...], (((1,), (1,)), ((), ())),
                        preferred_element_type=jnp.float32) + b2r[...]
    node = jax.nn.relu(_bn(y, g3[...], b3[...]))
    cat2[:, :D] = _mlp_bn_relu(node, w1s[...], b1s[...], g1s[...], be1s[...],
                               w2s[...], b2s[...], g4[...], b4[...])
    cat2[:, D:] = _mlp_bn_relu(node, w1d[...], b1d[...], g1d[...], be1d[...],
                               w2d[...], b2d[...], g5[...], b5[...])


def _pipelined(nchunks, start, drain, bufs):
    nbuf = len(bufs)
    assert nchunks % nbuf == 0
    for b in range(nbuf - 1):
        start(b, bufs[b])

    def body(g, _):
        j0 = g * nbuf
        for b in range(nbuf):
            i = j0 + b

            @pl.when(i + nbuf - 1 < nchunks)
            def _():
                start(i + nbuf - 1, bufs[(b + nbuf - 1) % nbuf])

            drain(i, bufs[b])
        return 0

    lax.fori_loop(0, nchunks // nbuf, body, 0)


_SC_MESH = plsc.VectorSubcoreMesh(core_axis_name="c", subcore_axis_name="s")


@functools.partial(
    pl.kernel,
    out_type=(jax.ShapeDtypeStruct((N, D), jnp.float32),
              jax.ShapeDtypeStruct((N, D), jnp.float32)),
    mesh=_SC_MESH,
    scratch_types=[
        pltpu.VMEM((GCH2, CH2), jnp.int32),
        pltpu.VMEM((GCH2, CH2), jnp.int32),
        pltpu.VMEM((GCH2, CH2), jnp.int32),
        pltpu.VMEM((GCH2, CH2), jnp.int32),
        pltpu.VMEM((CH2, D), jnp.float32),
        pltpu.VMEM((CH2, D), jnp.float32),
        pltpu.VMEM_SHARED((N, D), jnp.float32),
        pltpu.SemaphoreType.DMA,
        pltpu.SemaphoreType.DMA,
        pltpu.SemaphoreType.DMA,
        pltpu.SemaphoreType.DMA,
    ],
)
def _segment_sum_sc(srcf_hbm, dstf_hbm, zeros_hbm, src2d_hbm, dst2d_hbm,
                    aggA_hbm, aggB_hbm,
                    idxs0, idxd0, idxs1, idxd1, rows0, rows1, acc_sh,
                    sem0, sem1, isem0, isem1):
    c = lax.axis_index("c")
    s = lax.axis_index("s")
    zbuf_v = rows1.at[pl.ds(0, ZROWS)]

    def over_blocks(fn):
        def body(j, _):
            blk = s + j * NS

            @pl.when(blk < NBLOCKS)
            def _():
                fn(blk * ZROWS)

            return 0

        lax.fori_loop(0, (NBLOCKS + NS - 1) // NS, body, 0)

    idx_sets = ((idxs0, idxd0, isem0), (idxs1, idxd1, isem1))

    def idx_copies(g, iset):
        idxs, idxd, isem = iset
        base = s * TCH2 + g * GCH2
        return (pltpu.make_async_copy(
                    src2d_hbm.at[pl.ds(base, GCH2)], idxs, isem),
                pltpu.make_async_copy(
                    dst2d_hbm.at[pl.ds(base, GCH2)], idxd, isem))

    def load_idx(g, iset):
        for cp in idx_copies(g, iset):
            cp.start()

    def wait_idx(g, iset):
        for cp in idx_copies(g, iset):
            cp.wait()

    def run(table_hbm):
        def run_group(iset):
            idxs, idxd, _ = iset

            def start(i, bufsem):
                buf, sem = bufsem
                pltpu.async_copy(table_hbm.at[idxs.at[i]], buf, sem)

            def drain(i, bufsem):
                buf, sem = bufsem
                pltpu.make_async_copy(
                    table_hbm.at[idxs.at[i]], buf, sem).wait()
                pltpu.sync_copy(buf, acc_sh.at[idxd.at[i]], add=True)

            _pipelined(GCH2, start, drain, ((rows0, sem0), (rows1, sem1)))

        load_idx(0, idx_sets[0])

        def zinit(base):
            pltpu.sync_copy(zeros_hbm.at[pl.ds(base, ZROWS)], zbuf_v)
            pltpu.sync_copy(zbuf_v, acc_sh.at[pl.ds(base, ZROWS)])

        over_blocks(zinit)
        plsc.subcore_barrier()

        def group(g, _):
            for par in (0, 1):
                @pl.when(lax.rem(g, 2) == par)
                def _():
                    wait_idx(g, idx_sets[par])

                    @pl.when(g + 1 < NG2)
                    def _():
                        load_idx(g + 1, idx_sets[1 - par])

                    run_group(idx_sets[par])

            return 0

        lax.fori_loop(0, NG2, group, 0)

    @pl.when(c == 0)
    def _():
        run(srcf_hbm)

    @pl.when(c == 1)
    def _():
        run(dstf_hbm)

    plsc.subcore_barrier()

    def wb(out_hbm):
        def step(base):
            pltpu.sync_copy(acc_sh.at[pl.ds(base, ZROWS)], zbuf_v)
            pltpu.sync_copy(zbuf_v, out_hbm.at[pl.ds(base, ZROWS)])

        over_blocks(step)

    @pl.when(c == 0)
    def _():
        wb(aggA_hbm)

    @pl.when(c == 1)
    def _():
        wb(aggB_hbm)


@functools.partial(
    pl.kernel,
    out_type=jax.ShapeDtypeStruct((E, 2 * D), jnp.float32),
    mesh=_SC_MESH,
    scratch_types=[
        pltpu.VMEM((EPT4,), jnp.int32),
    ] + [pltpu.VMEM((CH4, 2 * D), jnp.float32)] * 5
      + [pltpu.SemaphoreType.DMA] * 5,
)
def _edge_gather_sc(cat2_hbm, src_hbm, out_hbm, idx_v, *bufsems):
    c = lax.axis_index("c")
    s = lax.axis_index("s")
    wid = s * NC + c
    e0 = wid * EPT4

    pltpu.sync_copy(src_hbm.at[pl.ds(e0, EPT4)], idx_v)

    def start(i, bufsem):
        buf, sem = bufsem
        pltpu.async_copy(cat2_hbm.at[idx_v.at[pl.ds(i * CH4, CH4)]], buf, sem)

    def drain(i, bufsem):
        buf, sem = bufsem
        pltpu.make_async_copy(
            cat2_hbm.at[idx_v.at[pl.ds(i * CH4, CH4)]], buf, sem).wait()
        pltpu.sync_copy(buf, out_hbm.at[pl.ds(e0 + i * CH4, CH4)])

    rows = bufsems[:5]
    sems = bufsems[5:]
    _pipelined(TCH4, start, drain, tuple(zip(rows, sems)))


def _mlp_args(p):
    return (p["W1"], p["b1"].reshape(1, D), p["g1"].reshape(1, D),
            p["be1"].reshape(1, D), p["W2"], p["b2"].reshape(1, D))


def kernel(feat, edge_index, params):
    src = edge_index[0]
    dst = edge_index[1]
    bn = params["bn"]

    def bnv(i):
        return (bn["g%d" % i].reshape(1, D), bn["b%d" % i].reshape(1, D))

    stage1_in = (feat, *_mlp_args(params["src1"]), *bnv(1),
                 *_mlp_args(params["dst1"]), *bnv(2))
    srcf, dstf, zeros = pl.pallas_call(
        _stage1_body,
        out_shape=(jax.ShapeDtypeStruct((N, D), jnp.float32),
                   jax.ShapeDtypeStruct((N, D), jnp.float32),
                   jax.ShapeDtypeStruct((N, D), jnp.float32)),
    )(*stage1_in)

    src2d = src.reshape(ROWS2, CH2)
    dst2d = dst.reshape(ROWS2, CH2)
    aggA, aggB = _segment_sum_sc(srcf, dstf, zeros, src2d, dst2d)

    pr = params["res"]
    stage3_in = (aggA, aggB, pr["W1"], pr["b1"].reshape(1, D),
                 pr["g1"].reshape(1, D), pr["be1"].reshape(1, D),
                 pr["W2"], pr["b2"].reshape(1, D), *bnv(3),
                 *_mlp_args(params["src2"]), *bnv(4),
                 *_mlp_args(params["dst2"]), *bnv(5))
    cat2 = pl.pallas_call(
        _stage3_body,
        out_shape=jax.ShapeDtypeStruct((N, 2 * D), jnp.float32),
    )(*stage3_in)

    return _edge_gather_sc(cat2, src)
